# baseline (speedup 1.0000x reference)
import functools

import jax
import jax.numpy as jnp
from jax import lax
from jax.experimental import pallas as pl
from jax.experimental.pallas import tpu as pltpu
from jax.experimental.pallas import tpu_sc as plsc

VOCAB = 1000
VPAD = 1024
LSE_PAD = 1024
BATCH, SEQ = 1024, 50
NTOK = BATCH * SEQ
NC, NS = 2, 16
NW = NC * NS
TPW = NTOK // NW
NBH = BATCH // 128
NVB = VPAD // 128
NITEM = SEQ * NBH * NVB
IPW = NITEM // NW
LCH = 80
NLCH = TPW // LCH


def _lse_body(tab_ref, out_ref):
    x = tab_ref[...]
    m = jnp.max(x, axis=1, keepdims=True)
    s = jnp.sum(jnp.exp(x - m), axis=1, keepdims=True)
    out_ref[...] = m + jnp.log(s)


_lse_call = pl.pallas_call(
    _lse_body,
    out_shape=jax.ShapeDtypeStruct((VOCAB, 1), jnp.float32),
)


def _tgather_body(table8_hbm, idx8_hbm, out5_hbm,
                  idxrows_v, idxr_v, in_v, os_v,
                  gsem0, gsem1, wsem0, wsem1):
    c = lax.axis_index("c")
    s = lax.axis_index("s")
    wid = s * NC + c
    kbase = wid * IPW
    t_lo = kbase // (NBH * NVB)

    pltpu.sync_copy(idx8_hbm.at[pl.ds(t_lo, 3)], idxrows_v)

    gsems = [gsem0, gsem1]
    wsems = [wsem0, wsem1]
    iotav = lax.iota(jnp.int32, 16)
    rid8 = [iotav + (blc * 16) for blc in range(8)]

    def decode(k):
        t = k // (NBH * NVB)
        r = k - t * (NBH * NVB)
        bh = r // NVB
        e = r - bh * NVB
        return t, bh, e

    def prep_idx(k, slot):
        t, bh, e = decode(k)
        tr = t - t_lo
        for h in range(8):
            row = idxrows_v[tr, pl.ds(bh * 128 + h * 16, 16)]
            idxr_v[slot, pl.ds(h * 16, 16)] = row + e

    def gather(slot, sem):
        return pltpu.make_async_copy(table8_hbm.at[idxr_v.at[slot]],
                                     in_v.at[slot], sem)

    def _wdesc16(k, slot, sem):
        t, bh, e = decode(k)
        return pltpu.make_async_copy(
            os_v.at[slot, pl.ds(0, 16)],
            out5_hbm.at[t, pl.ds(16 * e, 16), bh], sem)

    def _wdesc13(k, slot, sem):
        t, bh, _ = decode(k)
        return pltpu.make_async_copy(
            os_v.at[slot, pl.ds(0, 13)],
            out5_hbm.at[t, pl.ds(112, 13), bh], sem)

    def wstart(k, slot, sem):
        t, bh, e = decode(k)

        @pl.when(e < 7)
        def _():
            _wdesc16(k, slot, sem).start()

        @pl.when(e == 7)
        def _():
            _wdesc13(k, slot, sem).start()

    def wwait(k, slot, sem):
        t, bh, e = decode(k)

        @pl.when(e < 7)
        def _():
            _wdesc16(k, slot, sem).wait()

        @pl.when(e == 7)
        def _():
            _wdesc13(k, slot, sem).wait()

    prep_idx(kbase, 0)
    gather(0, gsem0).start()

    def pair_body(p, _):
        for b in range(2):
            k = kbase + p * 2 + b
            other = 1 - b

            @pl.when(p * 2 + b + 1 < IPW)
            def _():
                @pl.when(p * 2 + b >= 1)
                def _():
                    wwait(k - 1, other, wsems[other])
                prep_idx(k + 1, other)
                gather(other, gsems[other]).start()

            gather(b, gsems[b]).wait()

            def tbody(vhp, _2):
                for vl in range(8):
                    col = jnp.full((16,), vhp * 8 + vl, jnp.int32)
                    for blc in range(8):
                        v = plsc.load_gather(in_v.at[b], [rid8[blc], col])
                        os_v[b, vhp, vl, pl.ds(blc * 16, 16)] = v
                return 0

            lax.fori_loop(0, 16, tbody, 0)
            wstart(k, b, wsems[b])
        return 0

    lax.fori_loop(0, IPW // 2, pair_body, 0)
    wwait(kbase + IPW - 2, 0, wsem0)
    wwait(kbase + IPW - 1, 1, wsem1)


_tgather_call = functools.partial(
    pl.kernel,
    mesh=plsc.VectorSubcoreMesh(core_axis_name="c", subcore_axis_name="s"),
    compiler_params=pltpu.CompilerParams(use_tc_tiling_on_sc=False,
                                         needs_layout_passes=False),
    out_type=jax.ShapeDtypeStruct((SEQ, VOCAB // 8, NBH, 8, 128),
                                  jnp.float32),
    scratch_types=[
        pltpu.VMEM((3, BATCH), jnp.int32),
        pltpu.VMEM((2, 128), jnp.int32),
        pltpu.VMEM((2, 128, 128), jnp.float32),
        pltpu.VMEM((2, 16, 8, 128), jnp.float32),
        pltpu.SemaphoreType.DMA,
        pltpu.SemaphoreType.DMA,
        pltpu.SemaphoreType.DMA,
        pltpu.SemaphoreType.DMA,
    ],
)(_tgather_body)


def _loss_body(tabflat_hbm, idxflat_hbm, pickflat_hbm, lse_hbm, part_hbm,
               idx_v, pick_v, lse_v, pkv_v, acc_v, sem):
    c = lax.axis_index("c")
    s = lax.axis_index("s")
    wid = s * NC + c
    base = wid * TPW

    pltpu.sync_copy(idxflat_hbm.at[pl.ds(base, TPW)], idx_v)
    pltpu.sync_copy(pickflat_hbm.at[pl.ds(base, TPW)], pick_v)
    pltpu.sync_copy(lse_hbm, lse_v)

    def chunk_body(g, acc):
        pltpu.async_copy(tabflat_hbm.at[pick_v.at[pl.ds(g * LCH, LCH)]],
                         pkv_v, sem).wait()
        for h in range(LCH // 16):
            ixh = idx_v[pl.ds(g * LCH + h * 16, 16)]
            ls = plsc.load_gather(lse_v, [ixh])
            acc = acc + (ls - pkv_v[pl.ds(h * 16, 16)])
        return acc

    acc = lax.fori_loop(0, NLCH, chunk_body, jnp.zeros((16,), jnp.float32))
    acc_v[...] = acc
    pltpu.sync_copy(acc_v, part_hbm.at[wid])


_loss_call = functools.partial(
    pl.kernel,
    mesh=plsc.VectorSubcoreMesh(core_axis_name="c", subcore_axis_name="s"),
    compiler_params=pltpu.CompilerParams(use_tc_tiling_on_sc=False,
                                         needs_layout_passes=False),
    out_type=jax.ShapeDtypeStruct((NW, 16), jnp.float32),
    scratch_types=[
        pltpu.VMEM((TPW,), jnp.int32),
        pltpu.VMEM((TPW,), jnp.int32),
        pltpu.VMEM((LSE_PAD,), jnp.float32),
        pltpu.VMEM((LCH,), jnp.float32),
        pltpu.VMEM((16,), jnp.float32),
        pltpu.SemaphoreType.DMA,
    ],
)(_loss_body)


def _reduce_body(part_ref, out_ref):
    out_ref[...] = jnp.sum(part_ref[...], keepdims=True).reshape(1, 1) * (
        1.0 / NTOK)


_reduce_call = pl.pallas_call(
    _reduce_body,
    out_shape=jax.ShapeDtypeStruct((1, 1), jnp.float32),
)


@jax.jit
def kernel(idx, targets, table):
    idx32 = idx.astype(jnp.int32)
    tgt32 = targets.astype(jnp.int32)
    table_p = jnp.pad(table, ((0, 0), (0, VPAD - VOCAB)))
    table8 = table_p.reshape(VOCAB * 8, 128)
    idx8 = jnp.pad(idx32.T * 8, ((0, 6), (0, 0)))
    idxflat = idx32.reshape(-1)
    pickflat = idxflat * VPAD + tgt32.reshape(-1)
    lse = _lse_call(table)
    lse_p = jnp.pad(lse.reshape(-1), (0, LSE_PAD - VOCAB))
    out5 = _tgather_call(table8, idx8)
    logits = out5.transpose(2, 4, 0, 1, 3).reshape(BATCH, SEQ, VOCAB)
    parts = _loss_call(table_p.reshape(-1), idxflat, pickflat, lse_p)
    loss = _reduce_call(parts)[0, 0]
    return logits, loss


# speedup vs baseline: 1.6897x; 1.6897x over previous
import functools

import jax
import jax.numpy as jnp
from jax import lax
from jax.experimental import pallas as pl
from jax.experimental.pallas import tpu as pltpu
from jax.experimental.pallas import tpu_sc as plsc

VOCAB = 1000
VPAD = 1024
LSE_PAD = 1024
BATCH, SEQ = 1024, 50
NTOK = BATCH * SEQ
NC, NS = 2, 16
NW = NC * NS
BPW = BATCH // NW
TPW = NTOK // NW
SEQ_P = 56
LCH = 80
NLCH = TPW // LCH


def _lse_body(tab_ref, out_ref):
    x = tab_ref[...]
    m = jnp.max(x, axis=1, keepdims=True)
    s = jnp.sum(jnp.exp(x - m), axis=1, keepdims=True)
    out_ref[...] = m + jnp.log(s)


_lse_call = pl.pallas_call(
    _lse_body,
    out_shape=jax.ShapeDtypeStruct((VOCAB, 1), jnp.float32),
)


def _gather_body(table_hbm, idx_hbm, out_hbm, idx_v, rows_v,
                 gsem0, gsem1, wsem0, wsem1):
    c = lax.axis_index("c")
    s = lax.axis_index("s")
    wid = s * NC + c
    base = wid * BPW

    pltpu.sync_copy(idx_hbm.at[pl.ds(wid * (BPW * 64), BPW * 64)], idx_v)

    gsems = [gsem0, gsem1]
    wsems = [wsem0, wsem1]

    def gather(g, slot, sem):
        return pltpu.make_async_copy(
            table_hbm.at[idx_v.at[pl.ds(g * 64, SEQ_P)]], rows_v.at[slot],
            sem)

    def write(g, slot, sem):
        return pltpu.make_async_copy(rows_v.at[slot], out_hbm.at[base + g],
                                     sem)

    gather(0, 0, gsem0).start()

    def pair_body(t, _):
        for b in range(2):
            g = t * 2 + b
            other = 1 - b
            @pl.when(g + 1 < BPW)
            def _():
                @pl.when(g >= 1)
                def _():
                    write(g - 1, other, wsems[other]).wait()
                gather(g + 1, other, gsems[other]).start()
            gather(g, b, gsems[b]).wait()
            write(g, b, wsems[b]).start()
        return 0

    lax.fori_loop(0, BPW // 2, pair_body, 0)
    write(BPW - 2, 0, wsem0).wait()
    write(BPW - 1, 1, wsem1).wait()


_gather_call = functools.partial(
    pl.kernel,
    mesh=plsc.VectorSubcoreMesh(core_axis_name="c", subcore_axis_name="s"),
    compiler_params=pltpu.CompilerParams(use_tc_tiling_on_sc=True,
                                         needs_layout_passes=False),
    out_type=jax.ShapeDtypeStruct((BATCH, SEQ_P, VPAD), jnp.float32),
    scratch_types=[
        pltpu.VMEM((BPW * 64,), jnp.int32),
        pltpu.VMEM((2, SEQ_P, VPAD), jnp.float32),
        pltpu.SemaphoreType.DMA,
        pltpu.SemaphoreType.DMA,
        pltpu.SemaphoreType.DMA,
        pltpu.SemaphoreType.DMA,
    ],
)(_gather_body)


def _loss_body(tabflat_hbm, idxflat_hbm, pickflat_hbm, lse_hbm, part_hbm,
               idx_v, pick_v, lse_v, pkv_v, acc_v, sem):
    c = lax.axis_index("c")
    s = lax.axis_index("s")
    wid = s * NC + c
    base = wid * TPW

    pltpu.sync_copy(idxflat_hbm.at[pl.ds(base, TPW)], idx_v)
    pltpu.sync_copy(pickflat_hbm.at[pl.ds(base, TPW)], pick_v)
    pltpu.sync_copy(lse_hbm, lse_v)

    def chunk_body(g, acc):
        pltpu.async_copy(tabflat_hbm.at[pick_v.at[pl.ds(g * LCH, LCH)]],
                         pkv_v, sem).wait()
        for h in range(LCH // 16):
            ixh = idx_v[pl.ds(g * LCH + h * 16, 16)]
            ls = plsc.load_gather(lse_v, [ixh])
            acc = acc + (ls - pkv_v[pl.ds(h * 16, 16)])
        return acc

    acc = lax.fori_loop(0, NLCH, chunk_body, jnp.zeros((16,), jnp.float32))
    acc_v[...] = acc
    pltpu.sync_copy(acc_v, part_hbm.at[wid])


_loss_call = functools.partial(
    pl.kernel,
    mesh=plsc.VectorSubcoreMesh(core_axis_name="c", subcore_axis_name="s"),
    compiler_params=pltpu.CompilerParams(use_tc_tiling_on_sc=False,
                                         needs_layout_passes=False),
    out_type=jax.ShapeDtypeStruct((NW, 16), jnp.float32),
    scratch_types=[
        pltpu.VMEM((TPW,), jnp.int32),
        pltpu.VMEM((TPW,), jnp.int32),
        pltpu.VMEM((LSE_PAD,), jnp.float32),
        pltpu.VMEM((LCH,), jnp.float32),
        pltpu.VMEM((16,), jnp.float32),
        pltpu.SemaphoreType.DMA,
    ],
)(_loss_body)


def _reduce_body(part_ref, out_ref):
    out_ref[...] = jnp.sum(part_ref[...], keepdims=True).reshape(1, 1) * (
        1.0 / NTOK)


_reduce_call = pl.pallas_call(
    _reduce_body,
    out_shape=jax.ShapeDtypeStruct((1, 1), jnp.float32),
)


@jax.jit
def kernel(idx, targets, table):
    idx32 = idx.astype(jnp.int32)
    tgt32 = targets.astype(jnp.int32)
    table_p = jnp.pad(table, ((0, 0), (0, VPAD - VOCAB)))
    idxp1 = jnp.pad(idx32, ((0, 0), (0, 64 - SEQ))).reshape(-1)
    idxflat = idx32.reshape(-1)
    pickflat = idxflat * VPAD + tgt32.reshape(-1)
    lse = _lse_call(table)
    lse_p = jnp.pad(lse.reshape(-1), (0, LSE_PAD - VOCAB))
    logits = _gather_call(table_p, idxp1)[:, :SEQ, :VOCAB]
    parts = _loss_call(table_p.reshape(-1), idxflat, pickflat, lse_p)
    loss = _reduce_call(parts)[0, 0]
    return logits, loss


# speedup vs baseline: 3.6785x; 2.1770x over previous
import functools

import jax
import jax.numpy as jnp
from jax import lax
from jax.experimental import pallas as pl
from jax.experimental.pallas import tpu as pltpu
from jax.experimental.pallas import tpu_sc as plsc

VOCAB = 1000
VPAD = 1024
LSE_PAD = 1024
BATCH, SEQ = 1024, 50
NTOK = BATCH * SEQ
NC, NS = 2, 16
NW = NC * NS
BPW = BATCH // NW
TPW = NTOK // NW
SEQ_P = 56
LCH = 80
NLCH = TPW // LCH


def _lse_body(tab_ref, out_ref):
    x = tab_ref[...]
    m = jnp.max(x, axis=1, keepdims=True)
    s = jnp.sum(jnp.exp(x - m), axis=1, keepdims=True)
    out_ref[...] = m + jnp.log(s)


_lse_call = pl.pallas_call(
    _lse_body,
    out_shape=jax.ShapeDtypeStruct((VOCAB, 1), jnp.float32),
)


def _gather_body(table_hbm, idx_hbm, out_hbm, idx_v, rows_v,
                 gsem0, gsem1, wsem0, wsem1):
    c = lax.axis_index("c")
    s = lax.axis_index("s")
    wid = s * NC + c
    base = wid * BPW

    pltpu.sync_copy(idx_hbm.at[pl.ds(wid * (BPW * 64), BPW * 64)], idx_v)

    gsems = [gsem0, gsem1]
    wsems = [wsem0, wsem1]

    def gather(g, slot, sem):
        return pltpu.make_async_copy(
            table_hbm.at[idx_v.at[pl.ds(g * 64, SEQ_P)]], rows_v.at[slot],
            sem)

    def write(g, slot, sem):
        return pltpu.make_async_copy(rows_v.at[slot], out_hbm.at[base + g],
                                     sem)

    gather(0, 0, gsem0).start()

    def pair_body(t, _):
        for b in range(2):
            g = t * 2 + b
            other = 1 - b
            @pl.when(g + 1 < BPW)
            def _():
                @pl.when(g >= 1)
                def _():
                    write(g - 1, other, wsems[other]).wait()
                gather(g + 1, other, gsems[other]).start()
            gather(g, b, gsems[b]).wait()
            write(g, b, wsems[b]).start()
        return 0

    lax.fori_loop(0, BPW // 2, pair_body, 0)
    write(BPW - 2, 0, wsem0).wait()
    write(BPW - 1, 1, wsem1).wait()


_gather_call = functools.partial(
    pl.kernel,
    mesh=plsc.VectorSubcoreMesh(core_axis_name="c", subcore_axis_name="s"),
    compiler_params=pltpu.CompilerParams(use_tc_tiling_on_sc=True,
                                         needs_layout_passes=False),
    out_type=jax.ShapeDtypeStruct((BATCH, SEQ_P, VPAD), jnp.float32),
    scratch_types=[
        pltpu.VMEM((BPW * 64,), jnp.int32),
        pltpu.VMEM((2, SEQ_P, VPAD), jnp.float32),
        pltpu.SemaphoreType.DMA,
        pltpu.SemaphoreType.DMA,
        pltpu.SemaphoreType.DMA,
        pltpu.SemaphoreType.DMA,
    ],
)(_gather_body)


def _loss_body(tabflat_hbm, idxflat_hbm, pickflat_hbm, lse_hbm, part_hbm,
               idx_v, pick_v, lse_v, pkv_v, acc_v, sem):
    c = lax.axis_index("c")
    s = lax.axis_index("s")
    wid = s * NC + c
    base = wid * TPW

    pltpu.sync_copy(idxflat_hbm.at[pl.ds(base, TPW)], idx_v)
    pltpu.sync_copy(pickflat_hbm.at[pl.ds(base, TPW)], pick_v)
    pltpu.sync_copy(lse_hbm, lse_v)

    def chunk_body(g, acc):
        pltpu.async_copy(tabflat_hbm.at[pick_v.at[pl.ds(g * LCH, LCH)]],
                         pkv_v, sem).wait()
        for h in range(LCH // 16):
            ixh = idx_v[pl.ds(g * LCH + h * 16, 16)]
            ls = plsc.load_gather(lse_v, [ixh])
            acc = acc + (ls - pkv_v[pl.ds(h * 16, 16)])
        return acc

    acc = lax.fori_loop(0, NLCH, chunk_body, jnp.zeros((16,), jnp.float32))
    acc_v[...] = acc
    pltpu.sync_copy(acc_v, part_hbm.at[wid])


_loss_call = functools.partial(
    pl.kernel,
    mesh=plsc.VectorSubcoreMesh(core_axis_name="c", subcore_axis_name="s"),
    compiler_params=pltpu.CompilerParams(use_tc_tiling_on_sc=False,
                                         needs_layout_passes=False),
    out_type=jax.ShapeDtypeStruct((NW, 16), jnp.float32),
    scratch_types=[
        pltpu.VMEM((TPW,), jnp.int32),
        pltpu.VMEM((TPW,), jnp.int32),
        pltpu.VMEM((LSE_PAD,), jnp.float32),
        pltpu.VMEM((LCH,), jnp.float32),
        pltpu.VMEM((16,), jnp.float32),
        pltpu.SemaphoreType.DMA,
    ],
)(_loss_body)


def _reduce_body(part_ref, out_ref):
    out_ref[...] = jnp.sum(part_ref[...], keepdims=True).reshape(1, 1) * (
        1.0 / NTOK)


_reduce_call = pl.pallas_call(
    _reduce_body,
    out_shape=jax.ShapeDtypeStruct((1, 1), jnp.float32),
)


@jax.jit
def kernel(idx, targets, table):
    idx32 = idx.astype(jnp.int32)
    tgt32 = targets.astype(jnp.int32)
    table_p = jnp.pad(table, ((0, 0), (0, VPAD - VOCAB)))
    idxp1 = jnp.pad(idx32, ((0, 0), (0, 64 - SEQ)), mode="edge").reshape(-1)
    idxflat = idx32.reshape(-1)
    pickflat = idxflat * VPAD + tgt32.reshape(-1)
    lse = _lse_call(table)
    lse_p = jnp.pad(lse.reshape(-1), (0, LSE_PAD - VOCAB))
    logits = _gather_call(table_p, idxp1)[:, :SEQ, :VOCAB]
    parts = _loss_call(table_p.reshape(-1), idxflat, pickflat, lse_p)
    loss = _reduce_call(parts)[0, 0]
    return logits, loss
